# baseline (speedup 1.0000x reference)
import functools

import jax
import jax.numpy as jnp
from jax import lax
from jax.experimental import pallas as pl
from jax.experimental.pallas import tpu as pltpu
from jax.experimental.pallas import tpu_sc as plsc

N_EMBED = 128
BATCH = 4096
HIST = 50
NC = 2
NS = 16
NW = NC * NS
NCH = 4
CB = BATCH // NCH
BPW = CB // NW
K = 8
NSB = BPW // K
NBUF = 2
AB = 8

_mesh = plsc.VectorSubcoreMesh(core_axis_name="c", subcore_axis_name="s")


@functools.partial(
    pl.kernel,
    out_type=jax.ShapeDtypeStruct((CB * HIST, N_EMBED), jnp.float32),
    mesh=_mesh,
    scratch_types=[
        pltpu.VMEM((BPW, HIST), jnp.int32),
        [pltpu.VMEM((K * HIST, N_EMBED), jnp.float32) for _ in range(NBUF)],
        [pltpu.SemaphoreType.DMA for _ in range(NBUF)],
        [pltpu.SemaphoreType.DMA for _ in range(NBUF)],
    ],
)
def _gather_chunk(idx_hbm, table_hbm, out_hbm, idx_v, bufs, gsems, psems):
    wid = lax.axis_index("s") * NC + lax.axis_index("c")
    pltpu.sync_copy(idx_hbm.at[pl.ds(wid * BPW, BPW)], idx_v)
    fbase = wid * (BPW * HIST)

    def fire(s, r):
        for i in range(K):
            pltpu.async_copy(table_hbm.at[idx_v.at[s * K + i]],
                             bufs[r].at[pl.ds(i * HIST, HIST)], gsems[r])

    def drain(s, r):
        for i in range(K):
            pltpu.make_async_copy(table_hbm.at[idx_v.at[s * K + i]],
                                  bufs[r].at[pl.ds(i * HIST, HIST)],
                                  gsems[r]).wait()

    for r in range(NBUF):
        fire(r, r)

    def outer(t, carry):
        for r in range(NBUF):
            s = t * NBUF + r
            drain(s, r)
            pltpu.async_copy(
                bufs[r], out_hbm.at[pl.ds(fbase + s * (K * HIST), K * HIST)],
                psems[r]).wait()

            @pl.when(s + NBUF < NSB)
            def _():
                fire(s + NBUF, r)

        return carry

    lax.fori_loop(0, NSB // NBUF, outer, 0)


def _assemble_body(chunk_ref, acc_ref, out_ref):
    del acc_ref
    for i in range(AB):
        out_ref[i] = chunk_ref[pl.ds(i * HIST, HIST)]


def _assemble_first_body(chunk_ref, out_ref):
    for i in range(AB):
        out_ref[i] = chunk_ref[pl.ds(i * HIST, HIST)]


def _assemble(chunk, acc, c):
    grid = (CB // AB,)
    chunk_spec = pl.BlockSpec((AB * HIST, N_EMBED), lambda b: (b, 0))
    out_spec = pl.BlockSpec((AB, HIST, N_EMBED),
                            lambda b, _c0=c * (CB // AB): (b + _c0, 0, 0))
    if acc is None:
        return pl.pallas_call(
            _assemble_first_body,
            grid=grid,
            in_specs=[chunk_spec],
            out_specs=out_spec,
            out_shape=jax.ShapeDtypeStruct((BATCH, HIST, N_EMBED),
                                           jnp.float32),
        )(chunk)
    return pl.pallas_call(
        _assemble_body,
        grid=grid,
        in_specs=[chunk_spec,
                  pl.BlockSpec(memory_space=pltpu.MemorySpace.HBM)],
        out_specs=out_spec,
        out_shape=jax.ShapeDtypeStruct((BATCH, HIST, N_EMBED), jnp.float32),
        input_output_aliases={1: 0},
    )(chunk, acc)


def kernel(input, embedding):
    idx = input.astype(jnp.int32)
    chunks = [_gather_chunk(idx[c * CB:(c + 1) * CB], embedding)
              for c in range(NCH)]
    acc = None
    for c, ch in enumerate(chunks):
        acc = _assemble(ch, acc, c)
    return acc


# speedup vs baseline: 1.5287x; 1.5287x over previous
import functools

import jax
import jax.numpy as jnp
from jax import lax
from jax.experimental import pallas as pl
from jax.experimental.pallas import tpu as pltpu
from jax.experimental.pallas import tpu_sc as plsc

N_EMBED = 128
BATCH = 4096
HIST = 50
NC = 2
NS = 16
NW = NC * NS
NCH = 4
CB = BATCH // NCH
BPW = CB // NW
K = 8
NSB = BPW // K
NBUF = 2
AB = 8

_mesh = plsc.VectorSubcoreMesh(core_axis_name="c", subcore_axis_name="s")


@functools.partial(
    pl.kernel,
    out_type=jax.ShapeDtypeStruct((CB, HIST, N_EMBED), jnp.float32),
    mesh=_mesh,
    scratch_types=[
        pltpu.VMEM((BPW, HIST), jnp.int32),
        [pltpu.VMEM((K, HIST, N_EMBED), jnp.float32) for _ in range(NBUF)],
        [pltpu.SemaphoreType.DMA for _ in range(NBUF)],
        [pltpu.SemaphoreType.DMA for _ in range(NBUF)],
    ],
)
def _gather_chunk(idx_hbm, table_hbm, out_hbm, idx_v, bufs, gsems, psems):
    wid = lax.axis_index("s") * NC + lax.axis_index("c")
    b0 = wid * BPW
    pltpu.sync_copy(idx_hbm.at[pl.ds(b0, BPW)], idx_v)

    def fire(s, r):
        for i in range(K):
            pltpu.async_copy(table_hbm.at[idx_v.at[s * K + i]], bufs[r].at[i],
                             gsems[r])

    def drain(s, r):
        for i in range(K):
            pltpu.make_async_copy(table_hbm.at[idx_v.at[s * K + i]],
                                  bufs[r].at[i], gsems[r]).wait()

    for r in range(NBUF):
        fire(r, r)

    def outer(t, carry):
        for r in range(NBUF):
            s = t * NBUF + r
            drain(s, r)
            pltpu.async_copy(bufs[r], out_hbm.at[pl.ds(b0 + s * K, K)],
                             psems[r]).wait()

            @pl.when(s + NBUF < NSB)
            def _():
                fire(s + NBUF, r)

        return carry

    lax.fori_loop(0, NSB // NBUF, outer, 0)


def kernel(input, embedding):
    idx = input.astype(jnp.int32)
    chunks = [_gather_chunk(idx[c * CB:(c + 1) * CB], embedding)
              for c in range(NCH)]
    acc = jnp.zeros((BATCH, HIST, N_EMBED), jnp.float32)
    for c, ch in enumerate(chunks):
        acc = lax.dynamic_update_slice(acc, ch, (c * CB, 0, 0))
    return acc


# speedup vs baseline: 2.6938x; 1.7621x over previous
import functools

import jax
import jax.numpy as jnp
from jax import lax
from jax.experimental import pallas as pl
from jax.experimental.pallas import tpu as pltpu
from jax.experimental.pallas import tpu_sc as plsc

N_EMBED = 128
BATCH = 4096
HIST = 50
NC = 2
NS = 16
NW = NC * NS
NCH = 1
CB = BATCH // NCH
BPW = CB // NW
K = 4
NSB = BPW // K
NBUF = 4

_mesh = plsc.VectorSubcoreMesh(core_axis_name="c", subcore_axis_name="s")


@functools.partial(
    pl.kernel,
    out_type=jax.ShapeDtypeStruct((CB, HIST, N_EMBED), jnp.float32),
    mesh=_mesh,
    scratch_types=[
        pltpu.VMEM((BPW, HIST), jnp.int32),
        [pltpu.VMEM((K, HIST, N_EMBED), jnp.float32) for _ in range(NBUF)],
        [pltpu.SemaphoreType.DMA for _ in range(NBUF)],
        [pltpu.SemaphoreType.DMA for _ in range(NBUF)],
    ],
)
def _gather_chunk(idx_hbm, table_hbm, out_hbm, idx_v, bufs, gsems, psems):
    wid = lax.axis_index("s") * NC + lax.axis_index("c")
    b0 = wid * BPW
    pltpu.sync_copy(idx_hbm.at[pl.ds(b0, BPW)], idx_v)

    def fire(s, r):
        for i in range(K):
            pltpu.async_copy(table_hbm.at[idx_v.at[s * K + i]], bufs[r].at[i],
                             gsems[r])

    def drain(s, r):
        for i in range(K):
            pltpu.make_async_copy(table_hbm.at[idx_v.at[s * K + i]],
                                  bufs[r].at[i], gsems[r]).wait()

    def fire_put(s, r):
        return pltpu.async_copy(bufs[r], out_hbm.at[pl.ds(b0 + s * K, K)],
                                psems[r])

    def wait_put(s, r):
        pltpu.make_async_copy(bufs[r], out_hbm.at[pl.ds(b0 + s * K, K)],
                              psems[r]).wait()

    for r in range(NBUF - 1):
        fire(r, r)

    def outer(t, carry):
        for r in range(NBUF):
            s = t * NBUF + r
            drain(s, r)
            fire_put(s, r)

            @pl.when(s >= 1)
            def _():
                wait_put(s - 1, (r + NBUF - 1) % NBUF)

            @pl.when(s + NBUF - 1 < NSB)
            def _():
                fire(s + NBUF - 1, (r + NBUF - 1) % NBUF)

        return carry

    lax.fori_loop(0, NSB // NBUF, outer, 0)
    wait_put(NSB - 1, (NSB - 1) % NBUF)


def kernel(input, embedding):
    return _gather_chunk(input.astype(jnp.int32), embedding)
